# baseline (speedup 1.0000x reference)
import numpy as np
import jax
import jax.numpy as jnp
from jax.experimental import pallas as pl
from jax.experimental.pallas import tpu as pltpu

_TOKENS = 8192
_DM = 4096
_NE = 64
_NS = 8
_BS = 64
_BM = _NS * _BS

_UNIFORM_CONST = None


def _uniform_bits_np():
    n = _TOKENS * _NE
    idx = np.arange(n, dtype=np.uint32)
    ks0 = np.uint32(0)
    ks1 = np.uint32(1)
    ks2 = np.uint32(np.uint32(0x1BD11BDA) ^ ks0 ^ ks1)
    x0 = np.zeros(n, np.uint32) + ks0
    x1 = idx + ks1
    rot0 = (13, 15, 26, 6)
    rot1 = (17, 29, 16, 24)
    key_sched = ((ks1, ks2), (ks2, ks0), (ks0, ks1), (ks1, ks2), (ks2, ks0))
    rots = (rot0, rot1, rot0, rot1, rot0)
    for i in range(5):
        for d in rots[i]:
            x0 = x0 + x1
            x1 = x0 ^ ((x1 << np.uint32(d)) | (x1 >> np.uint32(32 - d)))
        ka, kb = key_sched[i]
        x0 = x0 + ka
        x1 = x1 + kb + np.uint32(i + 1)
    bits = x0 ^ x1
    float_bits = (bits >> np.uint32(9)) | np.uint32(0x3F800000)
    u = float_bits.view(np.float32) - np.float32(1.0)
    u = np.maximum(np.float32(0.0), u)
    return u.reshape(_TOKENS, _NE)


def _gate_onehot_kernel(*refs):
    x_refs = refs[:_NS]
    w_ref, u_ref, out_ref = refs[_NS], refs[_NS + 1], refs[_NS + 2]
    for j, x_ref in enumerate(x_refs):
        z = jax.lax.dot_general(
            x_ref[...], w_ref[...],
            dimension_numbers=(((1,), (1,)), ((), ())),
            preferred_element_type=jnp.float32,
        )
        eps = jnp.float32(1e-20)
        g = -jnp.log(-jnp.log(u_ref[pl.ds(j * _BS, _BS), :] + eps) + eps)
        z = z + g
        m = jnp.max(z, axis=1, keepdims=True)
        iota = jax.lax.broadcasted_iota(jnp.int32, z.shape, 1)
        cand = jnp.where(z >= m, iota, _NE)
        first = jnp.min(cand, axis=1, keepdims=True)
        out_ref[:, pl.ds(j * _BS, _BS)] = jnp.transpose(
            (iota == first).astype(jnp.float32))


def kernel(x, gate_weights):
    global _UNIFORM_CONST
    if _UNIFORM_CONST is None:
        _UNIFORM_CONST = _uniform_bits_np()
    u = jnp.asarray(_UNIFORM_CONST)
    xspecs = [
        pl.BlockSpec((_BS, _DM), lambda i, j=j: (_NS * i + j, 0))
        for j in range(_NS)
    ]
    out_t = pl.pallas_call(
        _gate_onehot_kernel,
        grid=(_TOKENS // _BM,),
        in_specs=xspecs + [
            pl.BlockSpec((_NE, _DM), lambda i: (0, 0)),
            pl.BlockSpec((_BM, _NE), lambda i: (i, 0)),
        ],
        out_specs=pl.BlockSpec((_NE, _BM), lambda i: (0, i)),
        out_shape=jax.ShapeDtypeStruct((_NE, _TOKENS), jnp.float32),
        compiler_params=pltpu.CompilerParams(
            dimension_semantics=(pltpu.PARALLEL,),
        ),
    )(*([x] * _NS), gate_weights, u)
    return jnp.transpose(out_t)


# speedup vs baseline: 1.0219x; 1.0219x over previous
import numpy as np
import jax
import jax.numpy as jnp
from jax.experimental import pallas as pl
from jax.experimental.pallas import tpu as pltpu

_TOKENS = 8192
_DM = 4096
_NE = 64
_NS = 8
_BS = 128
_BM = _NS * _BS

_UNIFORM_CONST = None


def _uniform_bits_np():
    n = _TOKENS * _NE
    idx = np.arange(n, dtype=np.uint32)
    ks0 = np.uint32(0)
    ks1 = np.uint32(1)
    ks2 = np.uint32(np.uint32(0x1BD11BDA) ^ ks0 ^ ks1)
    x0 = np.zeros(n, np.uint32) + ks0
    x1 = idx + ks1
    rot0 = (13, 15, 26, 6)
    rot1 = (17, 29, 16, 24)
    key_sched = ((ks1, ks2), (ks2, ks0), (ks0, ks1), (ks1, ks2), (ks2, ks0))
    rots = (rot0, rot1, rot0, rot1, rot0)
    for i in range(5):
        for d in rots[i]:
            x0 = x0 + x1
            x1 = x0 ^ ((x1 << np.uint32(d)) | (x1 >> np.uint32(32 - d)))
        ka, kb = key_sched[i]
        x0 = x0 + ka
        x1 = x1 + kb + np.uint32(i + 1)
    bits = x0 ^ x1
    float_bits = (bits >> np.uint32(9)) | np.uint32(0x3F800000)
    u = float_bits.view(np.float32) - np.float32(1.0)
    u = np.maximum(np.float32(0.0), u)
    return u.reshape(_TOKENS, _NE)


def _gate_onehot_kernel(*refs):
    x_refs = refs[:_NS]
    w_ref, u_ref, out_ref = refs[_NS], refs[_NS + 1], refs[_NS + 2]
    for j, x_ref in enumerate(x_refs):
        z = jax.lax.dot_general(
            x_ref[...], w_ref[...],
            dimension_numbers=(((1,), (1,)), ((), ())),
            preferred_element_type=jnp.float32,
        )
        eps = jnp.float32(1e-20)
        g = -jnp.log(-jnp.log(u_ref[pl.ds(j * _BS, _BS), :] + eps) + eps)
        z = z + g
        m = jnp.max(z, axis=1, keepdims=True)
        iota = jax.lax.broadcasted_iota(jnp.int32, z.shape, 1)
        cand = jnp.where(z >= m, iota, _NE)
        first = jnp.min(cand, axis=1, keepdims=True)
        out_ref[:, pl.ds(j * _BS, _BS)] = jnp.transpose(
            (iota == first).astype(jnp.float32))


def kernel(x, gate_weights):
    global _UNIFORM_CONST
    if _UNIFORM_CONST is None:
        _UNIFORM_CONST = _uniform_bits_np()
    u = jnp.asarray(_UNIFORM_CONST)
    xspecs = [
        pl.BlockSpec((_BS, _DM), lambda i, j=j: (_NS * i + j, 0))
        for j in range(_NS)
    ]
    out_t = pl.pallas_call(
        _gate_onehot_kernel,
        grid=(_TOKENS // _BM,),
        in_specs=xspecs + [
            pl.BlockSpec((_NE, _DM), lambda i: (0, 0)),
            pl.BlockSpec((_BM, _NE), lambda i: (i, 0)),
        ],
        out_specs=pl.BlockSpec((_NE, _BM), lambda i: (0, i)),
        out_shape=jax.ShapeDtypeStruct((_NE, _TOKENS), jnp.float32),
        compiler_params=pltpu.CompilerParams(
            dimension_semantics=(pltpu.PARALLEL,),
        ),
    )(*([x] * _NS), gate_weights, u)
    return jnp.transpose(out_t)


# speedup vs baseline: 1.0340x; 1.0118x over previous
import numpy as np
import jax
import jax.numpy as jnp
from jax.experimental import pallas as pl
from jax.experimental.pallas import tpu as pltpu

_TOKENS = 8192
_DM = 4096
_NE = 64
_NS = 4
_BS = 128
_BM = _NS * _BS

_UNIFORM_CONST = None


def _uniform_bits_np():
    n = _TOKENS * _NE
    idx = np.arange(n, dtype=np.uint32)
    ks0 = np.uint32(0)
    ks1 = np.uint32(1)
    ks2 = np.uint32(np.uint32(0x1BD11BDA) ^ ks0 ^ ks1)
    x0 = np.zeros(n, np.uint32) + ks0
    x1 = idx + ks1
    rot0 = (13, 15, 26, 6)
    rot1 = (17, 29, 16, 24)
    key_sched = ((ks1, ks2), (ks2, ks0), (ks0, ks1), (ks1, ks2), (ks2, ks0))
    rots = (rot0, rot1, rot0, rot1, rot0)
    for i in range(5):
        for d in rots[i]:
            x0 = x0 + x1
            x1 = x0 ^ ((x1 << np.uint32(d)) | (x1 >> np.uint32(32 - d)))
        ka, kb = key_sched[i]
        x0 = x0 + ka
        x1 = x1 + kb + np.uint32(i + 1)
    bits = x0 ^ x1
    float_bits = (bits >> np.uint32(9)) | np.uint32(0x3F800000)
    u = float_bits.view(np.float32) - np.float32(1.0)
    u = np.maximum(np.float32(0.0), u)
    return u.reshape(_TOKENS, _NE)


def _gate_onehot_kernel(*refs):
    x_refs = refs[:_NS]
    w_ref, u_ref, out_ref = refs[_NS], refs[_NS + 1], refs[_NS + 2]
    for j, x_ref in enumerate(x_refs):
        z = jax.lax.dot_general(
            x_ref[...], w_ref[...],
            dimension_numbers=(((1,), (1,)), ((), ())),
            preferred_element_type=jnp.float32,
        )
        eps = jnp.float32(1e-20)
        g = -jnp.log(-jnp.log(u_ref[pl.ds(j * _BS, _BS), :] + eps) + eps)
        z = z + g
        m = jnp.max(z, axis=1, keepdims=True)
        iota = jax.lax.broadcasted_iota(jnp.int32, z.shape, 1)
        cand = jnp.where(z >= m, iota, _NE)
        first = jnp.min(cand, axis=1, keepdims=True)
        out_ref[:, pl.ds(j * _BS, _BS)] = jnp.transpose(
            (iota == first).astype(jnp.float32))


def kernel(x, gate_weights):
    global _UNIFORM_CONST
    if _UNIFORM_CONST is None:
        _UNIFORM_CONST = _uniform_bits_np()
    u = jnp.asarray(_UNIFORM_CONST)
    xspecs = [
        pl.BlockSpec((_BS, _DM), lambda i, j=j: (_NS * i + j, 0))
        for j in range(_NS)
    ]
    out_t = pl.pallas_call(
        _gate_onehot_kernel,
        grid=(_TOKENS // _BM,),
        in_specs=xspecs + [
            pl.BlockSpec((_NE, _DM), lambda i: (0, 0)),
            pl.BlockSpec((_BM, _NE), lambda i: (i, 0)),
        ],
        out_specs=pl.BlockSpec((_NE, _BM), lambda i: (0, i)),
        out_shape=jax.ShapeDtypeStruct((_NE, _TOKENS), jnp.float32),
        compiler_params=pltpu.CompilerParams(
            dimension_semantics=(pltpu.PARALLEL,),
        ),
    )(*([x] * _NS), gate_weights, u)
    return jnp.transpose(out_t)


# speedup vs baseline: 1.0343x; 1.0002x over previous
import numpy as np
import jax
import jax.numpy as jnp
from jax.experimental import pallas as pl
from jax.experimental.pallas import tpu as pltpu

_TOKENS = 8192
_DM = 4096
_NE = 64
_NS = 4
_BS = 128
_BM = _NS * _BS

_UNIFORM_CONST = None


def _uniform_bits_np():
    n = _TOKENS * _NE
    idx = np.arange(n, dtype=np.uint32)
    ks0 = np.uint32(0)
    ks1 = np.uint32(1)
    ks2 = np.uint32(np.uint32(0x1BD11BDA) ^ ks0 ^ ks1)
    x0 = np.zeros(n, np.uint32) + ks0
    x1 = idx + ks1
    rot0 = (13, 15, 26, 6)
    rot1 = (17, 29, 16, 24)
    key_sched = ((ks1, ks2), (ks2, ks0), (ks0, ks1), (ks1, ks2), (ks2, ks0))
    rots = (rot0, rot1, rot0, rot1, rot0)
    for i in range(5):
        for d in rots[i]:
            x0 = x0 + x1
            x1 = x0 ^ ((x1 << np.uint32(d)) | (x1 >> np.uint32(32 - d)))
        ka, kb = key_sched[i]
        x0 = x0 + ka
        x1 = x1 + kb + np.uint32(i + 1)
    bits = x0 ^ x1
    float_bits = (bits >> np.uint32(9)) | np.uint32(0x3F800000)
    u = float_bits.view(np.float32) - np.float32(1.0)
    u = np.maximum(np.float32(0.0), u)
    return u.reshape(_TOKENS, _NE)


_REGION = _TOKENS // _NS


def _gate_onehot_kernel(*refs):
    x_refs = refs[:_NS]
    w_ref, u_ref, out_ref = refs[_NS], refs[_NS + 1], refs[_NS + 2]
    i = pl.program_id(0)
    for j, x_ref in enumerate(x_refs):
        row0 = j * _REGION + i * _BS
        z = jax.lax.dot_general(
            x_ref[...], w_ref[...],
            dimension_numbers=(((1,), (1,)), ((), ())),
            preferred_element_type=jnp.float32,
        )
        eps = jnp.float32(1e-20)
        g = -jnp.log(-jnp.log(u_ref[pl.ds(row0, _BS), :] + eps) + eps)
        z = z + g
        m = jnp.max(z, axis=1, keepdims=True)
        iota = jax.lax.broadcasted_iota(jnp.int32, z.shape, 1)
        cand = jnp.where(z >= m, iota, _NE)
        first = jnp.min(cand, axis=1, keepdims=True)
        out_ref[:, pl.ds(row0, _BS)] = jnp.transpose(
            (iota == first).astype(jnp.float32))


def kernel(x, gate_weights):
    global _UNIFORM_CONST
    if _UNIFORM_CONST is None:
        _UNIFORM_CONST = _uniform_bits_np()
    u = jnp.asarray(_UNIFORM_CONST)
    xspecs = [
        pl.BlockSpec((_BS, _DM), lambda i, j=j: (j * (_REGION // _BS) + i, 0))
        for j in range(_NS)
    ]
    out_t = pl.pallas_call(
        _gate_onehot_kernel,
        grid=(_TOKENS // _BM,),
        in_specs=xspecs + [
            pl.BlockSpec((_NE, _DM), lambda i: (0, 0)),
            pl.BlockSpec((_TOKENS, _NE), lambda i: (0, 0)),
        ],
        out_specs=pl.BlockSpec((_NE, _TOKENS), lambda i: (0, 0)),
        out_shape=jax.ShapeDtypeStruct((_NE, _TOKENS), jnp.float32),
        compiler_params=pltpu.CompilerParams(
            dimension_semantics=(pltpu.PARALLEL,),
        ),
    )(*([x] * _NS), gate_weights, u)
    return jnp.transpose(out_t)


# speedup vs baseline: 1.0389x; 1.0045x over previous
import numpy as np
import jax
import jax.numpy as jnp
from jax.experimental import pallas as pl
from jax.experimental.pallas import tpu as pltpu

_TOKENS = 8192
_DM = 4096
_NE = 64
_NS = 4
_BS = 128
_BM = _NS * _BS

_UNIFORM_CONST = None


def _uniform_bits_np():
    n = _TOKENS * _NE
    idx = np.arange(n, dtype=np.uint32)
    ks0 = np.uint32(0)
    ks1 = np.uint32(1)
    ks2 = np.uint32(np.uint32(0x1BD11BDA) ^ ks0 ^ ks1)
    x0 = np.zeros(n, np.uint32) + ks0
    x1 = idx + ks1
    rot0 = (13, 15, 26, 6)
    rot1 = (17, 29, 16, 24)
    key_sched = ((ks1, ks2), (ks2, ks0), (ks0, ks1), (ks1, ks2), (ks2, ks0))
    rots = (rot0, rot1, rot0, rot1, rot0)
    for i in range(5):
        for d in rots[i]:
            x0 = x0 + x1
            x1 = x0 ^ ((x1 << np.uint32(d)) | (x1 >> np.uint32(32 - d)))
        ka, kb = key_sched[i]
        x0 = x0 + ka
        x1 = x1 + kb + np.uint32(i + 1)
    bits = x0 ^ x1
    float_bits = (bits >> np.uint32(9)) | np.uint32(0x3F800000)
    u = float_bits.view(np.float32) - np.float32(1.0)
    u = np.maximum(np.float32(0.0), u)
    return u.reshape(_TOKENS, _NE)


def _gate_onehot_kernel(*refs):
    x_refs = refs[:_NS]
    w_ref, u_ref, out_ref = refs[_NS], refs[_NS + 1], refs[_NS + 2]
    for j, x_ref in enumerate(x_refs):
        z = jax.lax.dot_general(
            x_ref[...], w_ref[...],
            dimension_numbers=(((1,), (1,)), ((), ())),
            preferred_element_type=jnp.float32,
        )
        eps = jnp.float32(1e-20)
        g = -jnp.log(-jnp.log(u_ref[pl.ds(j * _BS, _BS), :] + eps) + eps)
        z = z + g
        m = jnp.max(z, axis=1, keepdims=True)
        iota = jax.lax.broadcasted_iota(jnp.int32, z.shape, 1)
        cand = jnp.where(z >= m, iota, _NE)
        first = jnp.min(cand, axis=1, keepdims=True)
        out_ref[:, pl.ds(j * _BS, _BS)] = jnp.transpose(
            (iota == first).astype(jnp.float32))


def kernel(x, gate_weights):
    global _UNIFORM_CONST
    if _UNIFORM_CONST is None:
        _UNIFORM_CONST = _uniform_bits_np()
    u = jnp.asarray(_UNIFORM_CONST)
    xspecs = [
        pl.BlockSpec((_BS, _DM), lambda i, j=j: (_NS * i + j, 0))
        for j in range(_NS)
    ]
    out_t = pl.pallas_call(
        _gate_onehot_kernel,
        grid=(_TOKENS // _BM,),
        in_specs=xspecs + [
            pl.BlockSpec((_NE, _DM), lambda i: (0, 0)),
            pl.BlockSpec((_BM, _NE), lambda i: (i, 0)),
        ],
        out_specs=pl.BlockSpec((_NE, _BM), lambda i: (0, i)),
        out_shape=jax.ShapeDtypeStruct((_NE, _TOKENS), jnp.float32),
        compiler_params=pltpu.CompilerParams(
            dimension_semantics=(pltpu.PARALLEL,),
        ),
    )(*([x] * _NS), gate_weights, u)
    return jnp.transpose(out_t)
